# baseline (speedup 1.0000x reference)
import jax
import jax.numpy as jnp
from jax.experimental import pallas as pl
from jax.experimental.pallas import tpu as pltpu
from jax import lax

E = 1024
H = 8
D = 8
C = 64
F32 = jnp.float32

_IN_SHAPES = (
    (1, 3 * E),
    (E, 123),
    (3 * E, C),
    (1, C),
    (3, C),
    (1, C),
    (6, C),
    (1, C),
    (6, C),
    (1, C),
    (1, C), (1, C),
    (1, C), (1, C),
    (1, C), (1, C),
    (C, C),
    (1, C),
    (1, C),
    (C, C),
    (1, C),
    (C, C),
    (1, C),
    (1, C), (1, C),
)
_N_IN = len(_IN_SHAPES)


def _leaky(a):
    return jnp.where(a >= 0.0, a, 0.2 * a)


def _dot(a, b):
    return jnp.dot(a, b, preferred_element_type=F32)


def _han_body(*refs):
    hbm = refs[:_N_IN]
    out_ref = refs[_N_IN]
    vmem = refs[_N_IN + 1: 2 * _N_IN + 1]
    sems = refs[2 * _N_IN + 1]

    copies = [pltpu.make_async_copy(hbm[i], vmem[i], sems.at[i])
              for i in range(_N_IN)]
    for cp in copies:
        cp.start()
    for cp in copies:
        cp.wait()

    (xA_ref, xR_ref,
     WA_ref, bA_ref, WE_ref, bE_ref, WR_ref, bR_ref, WW_ref, bW_ref,
     lsEA_ref, ldEA_ref, lsRE_ref, ldRE_ref, lsWE_ref, ldWE_ref,
     Wk_ref, bk_ref, q_ref,
     W2E_ref, b2E_ref, W2A_ref, b2A_ref, l2s_ref, l2d_ref) = vmem

    xA = xA_ref[...]
    xR = xR_ref[...]

    r8_rows = lax.broadcasted_iota(jnp.int32, (H, C), 0)
    r8_cols = lax.broadcasted_iota(jnp.int32, (H, C), 1)
    R8 = (r8_cols // D == r8_rows).astype(F32)
    t_rows = lax.broadcasted_iota(jnp.int32, (C, H), 0)
    t_cols = lax.broadcasted_iota(jnp.int32, (C, H), 1)
    R8T = (t_rows // D == t_cols).astype(F32)

    def head_sum(xn, lin_row):
        return _dot(xn * lin_row, R8T)

    xnA = _dot(xA, WA_ref[...]) + bA_ref[...]
    xnE = _dot(xR[:, 0:3], WE_ref[...]) + bE_ref[...]

    def slot_agg(col0, Wp_ref, bp_ref, ls_ref, a_dst):
        Wp = Wp_ref[...]
        bp = bp_ref[...]
        ls = ls_ref[...]
        xns, alphas = [], []
        for j in range(10):
            feat = xR[:, col0 + 6 * j: col0 + 6 * j + 6]
            xnj = _dot(feat, Wp) + bp
            active = jnp.sum(feat, axis=1, keepdims=True) != 0.0
            al = _leaky(head_sum(xnj, ls) + a_dst)
            al = jnp.where(active, al, -jnp.inf)
            xns.append(xnj)
            alphas.append(al)
        amax = alphas[0]
        for j in range(1, 10):
            amax = jnp.maximum(amax, alphas[j])
        amax = jnp.where(jnp.isfinite(amax), amax, 0.0)
        exs = [jnp.exp(a - amax) for a in alphas]
        s = exs[0]
        for j in range(1, 10):
            s = s + exs[j]
        inv = 1.0 / (s + 1e-16)
        agg = _dot(exs[0] * inv, R8) * xns[0]
        for j in range(1, 10):
            agg = agg + _dot(exs[j] * inv, R8) * xns[j]
        return jnp.maximum(agg, 0.0)

    ch_RE = slot_agg(3, WR_ref, bR_ref, lsRE_ref, head_sum(xnE, ldRE_ref[...]))
    ch_WE = slot_agg(63, WW_ref, bW_ref, lsWE_ref, head_sum(xnE, ldWE_ref[...]))
    ch_AE = jnp.maximum(xnA, 0.0)

    alEA = _leaky(head_sum(xnE, lsEA_ref[...]) + head_sum(xnA, ldEA_ref[...]))
    amax = jnp.max(alEA, axis=0, keepdims=True)
    ex = jnp.exp(alEA - amax)
    attn = ex / (jnp.sum(ex, axis=0, keepdims=True) + 1e-16)
    res1A = jnp.maximum(
        jnp.sum(_dot(attn, R8) * xnE, axis=0, keepdims=True), 0.0)

    Wk = Wk_ref[...]
    bk = bk_ref[...]
    q = q_ref[...]
    t0 = jnp.tanh(_dot(ch_AE, Wk) + bk)
    s0 = jnp.sum(t0 * q, axis=1, keepdims=True)
    t1 = jnp.mean(jnp.tanh(_dot(ch_RE, Wk) + bk), axis=0, keepdims=True)
    s1 = jnp.sum(t1 * q, axis=1, keepdims=True)
    t2 = jnp.mean(jnp.tanh(_dot(ch_WE, Wk) + bk), axis=0, keepdims=True)
    s2 = jnp.sum(t2 * q, axis=1, keepdims=True)
    m = jnp.maximum(jnp.maximum(s0, s1), s2)
    e0 = jnp.exp(s0 - m)
    e1 = jnp.exp(s1 - m)
    e2 = jnp.exp(s2 - m)
    invz = 1.0 / (e0 + e1 + e2)
    res1E = (e0 * invz) * ch_AE + (e1 * invz) * ch_RE + (e2 * invz) * ch_WE

    xn2E = _dot(res1E, W2E_ref[...]) + b2E_ref[...]
    xn2A = _dot(res1A, W2A_ref[...]) + b2A_ref[...]
    al2 = _leaky(head_sum(xn2E, l2s_ref[...]) + head_sum(xn2A, l2d_ref[...]))
    amax2 = jnp.max(al2, axis=0, keepdims=True)
    ex2 = jnp.exp(al2 - amax2)
    attn2 = ex2 / (jnp.sum(ex2, axis=0, keepdims=True) + 1e-16)
    agg2 = jnp.sum(_dot(attn2, R8) * xn2E, axis=0, keepdims=True)
    out_ref[...] = jnp.maximum(agg2, 0.0)


def kernel(x, params):
    x_flat = x.reshape(-1)
    xA = x_flat[: 3 * E].reshape(1, 3 * E)
    xR = x_flat[3 * E:].reshape(E, 123)

    p1 = params['conv1']
    p2 = params['conv2']

    def row(v):
        return v.reshape(1, C)

    args = (
        xA, xR,
        p1['proj']['Arrived']['W'], row(p1['proj']['Arrived']['b']),
        p1['proj']['Expert']['W'], row(p1['proj']['Expert']['b']),
        p1['proj']['Running']['W'], row(p1['proj']['Running']['b']),
        p1['proj']['Waiting']['W'], row(p1['proj']['Waiting']['b']),
        row(p1['lin_src']['Expert__Arrived']), row(p1['lin_dst']['Expert__Arrived']),
        row(p1['lin_src']['Running__Expert']), row(p1['lin_dst']['Running__Expert']),
        row(p1['lin_src']['Waiting__Expert']), row(p1['lin_dst']['Waiting__Expert']),
        p1['k_lin']['W'], row(p1['k_lin']['b']), row(p1['q']),
        p2['proj']['Expert']['W'], row(p2['proj']['Expert']['b']),
        p2['proj']['Arrived']['W'], row(p2['proj']['Arrived']['b']),
        row(p2['lin_src']['Expert__Arrived']), row(p2['lin_dst']['Expert__Arrived']),
    )

    return pl.pallas_call(
        _han_body,
        out_shape=jax.ShapeDtypeStruct((1, C), F32),
        in_specs=[pl.BlockSpec(memory_space=pl.ANY)] * _N_IN,
        scratch_shapes=(
            [pltpu.VMEM(s, F32) for s in _IN_SHAPES]
            + [pltpu.SemaphoreType.DMA((_N_IN,))]
        ),
    )(*args)


# speedup vs baseline: 1.1609x; 1.1609x over previous
import jax
import jax.numpy as jnp
from jax.experimental import pallas as pl
from jax import lax

E = 1024
H = 8
D = 8
C = 64
F32 = jnp.float32

_R_BA, _R_BE, _R_BR, _R_BW, _R_BK, _R_B2E, _R_B2A, _R_Q = range(8)
_R_LS_EA, _R_LD_EA, _R_LS_RE, _R_LD_RE, _R_LS_WE, _R_LD_WE, _R_L2S, _R_L2D = range(8, 16)
_R_WE = 16
_R_WR = 24
_R_WW = 32
_R_WK = 40
_R_W2E = 104
_R_W2A = 168
_PACK_ROWS = 232


def _leaky(a):
    return jnp.where(a >= 0.0, a, 0.2 * a)


def _dot(a, b):
    return jnp.dot(a, b, preferred_element_type=F32)


def _han_body(x_ref, xR_ref, WA_ref, pk_ref, out_ref):
    xA = x_ref[:, 0: 3 * E]
    xR = xR_ref[...]

    def prow(r):
        return pk_ref[r:r + 1, :]

    r8_rows = lax.broadcasted_iota(jnp.int32, (H, C), 0)
    r8_cols = lax.broadcasted_iota(jnp.int32, (H, C), 1)
    R8 = (r8_cols // D == r8_rows).astype(F32)
    t_rows = lax.broadcasted_iota(jnp.int32, (C, H), 0)
    t_cols = lax.broadcasted_iota(jnp.int32, (C, H), 1)
    R8T = (t_rows // D == t_cols).astype(F32)

    def head_sum(xn, lin_row):
        return _dot(xn * lin_row, R8T)

    xnA = _dot(xA, WA_ref[...]) + prow(_R_BA)
    xnE = _dot(xR[:, 0:3], pk_ref[_R_WE:_R_WE + 3, :]) + prow(_R_BE)

    def slot_agg(col0, w_row, b_row, ls_row, a_dst):
        Wp = pk_ref[w_row:w_row + 6, :]
        bp = prow(b_row)
        ls = prow(ls_row)
        xns, alphas = [], []
        for j in range(10):
            feat = xR[:, col0 + 6 * j: col0 + 6 * j + 6]
            xnj = _dot(feat, Wp) + bp
            active = jnp.sum(feat, axis=1, keepdims=True) != 0.0
            al = _leaky(head_sum(xnj, ls) + a_dst)
            al = jnp.where(active, al, -jnp.inf)
            xns.append(xnj)
            alphas.append(al)
        amax = alphas[0]
        for j in range(1, 10):
            amax = jnp.maximum(amax, alphas[j])
        amax = jnp.where(jnp.isfinite(amax), amax, 0.0)
        exs = [jnp.exp(a - amax) for a in alphas]
        s = exs[0]
        for j in range(1, 10):
            s = s + exs[j]
        inv = 1.0 / (s + 1e-16)
        agg = _dot(exs[0] * inv, R8) * xns[0]
        for j in range(1, 10):
            agg = agg + _dot(exs[j] * inv, R8) * xns[j]
        return jnp.maximum(agg, 0.0)

    ch_RE = slot_agg(3, _R_WR, _R_BR, _R_LS_RE, head_sum(xnE, prow(_R_LD_RE)))
    ch_WE = slot_agg(63, _R_WW, _R_BW, _R_LS_WE, head_sum(xnE, prow(_R_LD_WE)))
    ch_AE = jnp.maximum(xnA, 0.0)

    alEA = _leaky(head_sum(xnE, prow(_R_LS_EA)) + head_sum(xnA, prow(_R_LD_EA)))
    amax = jnp.max(alEA, axis=0, keepdims=True)
    ex = jnp.exp(alEA - amax)
    attn = ex / (jnp.sum(ex, axis=0, keepdims=True) + 1e-16)
    res1A = jnp.maximum(
        jnp.sum(_dot(attn, R8) * xnE, axis=0, keepdims=True), 0.0)

    Wk = pk_ref[_R_WK:_R_WK + C, :]
    bk = prow(_R_BK)
    q = prow(_R_Q)
    t0 = jnp.tanh(_dot(ch_AE, Wk) + bk)
    s0 = jnp.sum(t0 * q, axis=1, keepdims=True)
    t1 = jnp.mean(jnp.tanh(_dot(ch_RE, Wk) + bk), axis=0, keepdims=True)
    s1 = jnp.sum(t1 * q, axis=1, keepdims=True)
    t2 = jnp.mean(jnp.tanh(_dot(ch_WE, Wk) + bk), axis=0, keepdims=True)
    s2 = jnp.sum(t2 * q, axis=1, keepdims=True)
    m = jnp.maximum(jnp.maximum(s0, s1), s2)
    e0 = jnp.exp(s0 - m)
    e1 = jnp.exp(s1 - m)
    e2 = jnp.exp(s2 - m)
    invz = 1.0 / (e0 + e1 + e2)
    res1E = (e0 * invz) * ch_AE + (e1 * invz) * ch_RE + (e2 * invz) * ch_WE

    xn2E = _dot(res1E, pk_ref[_R_W2E:_R_W2E + C, :]) + prow(_R_B2E)
    xn2A = _dot(res1A, pk_ref[_R_W2A:_R_W2A + C, :]) + prow(_R_B2A)
    al2 = _leaky(head_sum(xn2E, prow(_R_L2S)) + head_sum(xn2A, prow(_R_L2D)))
    amax2 = jnp.max(al2, axis=0, keepdims=True)
    ex2 = jnp.exp(al2 - amax2)
    attn2 = ex2 / (jnp.sum(ex2, axis=0, keepdims=True) + 1e-16)
    agg2 = jnp.sum(_dot(attn2, R8) * xn2E, axis=0, keepdims=True)
    out_ref[...] = jnp.maximum(agg2, 0.0)


def kernel(x, params):
    x2d = x.reshape(1, -1)
    xR = x.reshape(-1)[3 * E:].reshape(E, 123)

    p1 = params['conv1']
    p2 = params['conv2']

    def row(v):
        return v.reshape(1, C)

    def put(a, r):
        return jnp.pad(a, ((r, _PACK_ROWS - r - a.shape[0]), (0, C - a.shape[1])))

    pieces = [
        (row(p1['proj']['Arrived']['b']), _R_BA),
        (row(p1['proj']['Expert']['b']), _R_BE),
        (row(p1['proj']['Running']['b']), _R_BR),
        (row(p1['proj']['Waiting']['b']), _R_BW),
        (row(p1['k_lin']['b']), _R_BK),
        (row(p2['proj']['Expert']['b']), _R_B2E),
        (row(p2['proj']['Arrived']['b']), _R_B2A),
        (row(p1['q']), _R_Q),
        (row(p1['lin_src']['Expert__Arrived']), _R_LS_EA),
        (row(p1['lin_dst']['Expert__Arrived']), _R_LD_EA),
        (row(p1['lin_src']['Running__Expert']), _R_LS_RE),
        (row(p1['lin_dst']['Running__Expert']), _R_LD_RE),
        (row(p1['lin_src']['Waiting__Expert']), _R_LS_WE),
        (row(p1['lin_dst']['Waiting__Expert']), _R_LD_WE),
        (row(p2['lin_src']['Expert__Arrived']), _R_L2S),
        (row(p2['lin_dst']['Expert__Arrived']), _R_L2D),
        (p1['proj']['Expert']['W'], _R_WE),
        (p1['proj']['Running']['W'], _R_WR),
        (p1['proj']['Waiting']['W'], _R_WW),
        (p1['k_lin']['W'], _R_WK),
        (p2['proj']['Expert']['W'], _R_W2E),
        (p2['proj']['Arrived']['W'], _R_W2A),
    ]
    pack = put(*pieces[0])
    for a, r in pieces[1:]:
        pack = pack + put(a, r)

    return pl.pallas_call(
        _han_body,
        out_shape=jax.ShapeDtypeStruct((1, C), F32),
    )(x2d, xR, p1['proj']['Arrived']['W'], pack)
